# Initial kernel scaffold; baseline (speedup 1.0000x reference)
import jax
import jax.numpy as jnp
from jax.experimental import pallas as pl


def kernel(x, expert_weights, W1, W2, top_experts):
    raise NotImplementedError("write your pallas kernel here")


# speedup vs baseline: 2.6968x; 2.6968x over previous
import functools

import jax
import jax.numpy as jnp
from jax import lax
from jax.experimental import pallas as pl
from jax.experimental.pallas import tpu as pltpu
from jax.experimental.pallas import tpu_sc as plsc

E = 8
T = 4096
K = 2
TK = T * K
D = 1024
F = 4096
BLK = 256
NB = (TK + E * (BLK - 1) + BLK - 1) // BLK
P = NB * BLK
FT = 1024
NF = F // FT
BE_PAD = 64

_info = plsc.get_sparse_core_info()
NC = _info.num_cores
NS = _info.num_subcores
L = _info.num_lanes

CHUNK = TK // NS
NV = CHUNK // L
TOKC = T // NS
NW = NC * NS
TPW = T // NW

_mesh = plsc.VectorSubcoreMesh(core_axis_name="c", subcore_axis_name="s")


@functools.partial(
    pl.kernel,
    mesh=_mesh,
    compiler_params=pltpu.CompilerParams(needs_layout_passes=False),
    out_type=[
        jax.ShapeDtypeStruct((P, D), jnp.float32),
        jax.ShapeDtypeStruct((P,), jnp.float32),
        jax.ShapeDtypeStruct((BE_PAD,), jnp.int32),
        jax.ShapeDtypeStruct((TK,), jnp.int32),
        jax.ShapeDtypeStruct((L,), jnp.int32),
    ],
    scratch_types=[
        pltpu.VMEM((CHUNK,), jnp.int32),
        pltpu.VMEM((4, 128), jnp.float32),
        pltpu.VMEM((CHUNK,), jnp.int32),
        pltpu.VMEM((4, 128), jnp.int32),
        pltpu.VMEM((4, 64), jnp.int32),
        pltpu.VMEM((4, 64), jnp.int32),
        pltpu.VMEM((64, D), jnp.float32),
        pltpu.VMEM((L,), jnp.int32),
        pltpu.VMEM((NS, L), jnp.int32),
        pltpu.VMEM_SHARED((NS, L), jnp.int32),
        pltpu.VMEM((BE_PAD,), jnp.int32),
        pltpu.VMEM((L,), jnp.int32),
        pltpu.SemaphoreType.DMA,
    ],
)
def _route(x_hbm, fe_hbm, ew_hbm, xg_hbm, gw_hbm, be_hbm, pos_hbm, cnt_hbm,
           e_chunk, w4, posv, pos4, pe, po, xrows, hist_my, hist_all,
           hist_stage, be_v, cnt_v, sem):
    cid = lax.axis_index("c")
    sid = lax.axis_index("s")
    base = sid * CHUNK
    iota = lax.iota(jnp.int32, L)

    pltpu.sync_copy(fe_hbm.at[pl.ds(base, CHUNK)], e_chunk)
    pltpu.sync_copy(ew_hbm.at[pl.ds(sid * 4, 4)], w4)

    c_acc = [jnp.int32(0) for _ in range(E)]
    for i in range(NV):
        v = e_chunk[pl.ds(i * L, L)]
        for e in range(E):
            c_acc[e] = c_acc[e] + jnp.sum((v == e).astype(jnp.int32))
    hv = jnp.zeros((L,), jnp.int32)
    for e in range(E):
        hv = jnp.where(iota == e, c_acc[e], hv)
    hist_my[...] = hv
    pltpu.sync_copy(hist_my, hist_stage.at[sid])
    plsc.subcore_barrier()
    pltpu.sync_copy(hist_stage, hist_all)

    hrow = [hist_all[s, pl.ds(0, L)] for s in range(NS)]
    counts = []
    pref = []
    for e in range(E):
        tot = hrow[0][e]
        for s in range(1, NS):
            tot = tot + hrow[s][e]
        counts.append(tot)
        pr = jnp.int32(0)
        for s in range(NS):
            pr = pr + jnp.where(jnp.int32(s) < sid, hrow[s][e], 0)
        pref.append(pr)
    padded = [((counts[e] + (BLK - 1)) >> 8) << 8 for e in range(E)]
    excl = []
    cum = []
    run = jnp.int32(0)
    for e in range(E):
        excl.append(run)
        run = run + padded[e]
        cum.append(run)
    start = [excl[e] + pref[e] for e in range(E)]

    for j in range(BE_PAD // L):
        bv = (iota + j * L) * BLK
        acc = jnp.zeros((L,), jnp.int32)
        for e in range(E):
            acc = acc + (cum[e] <= bv).astype(jnp.int32)
        be_v[pl.ds(j * L, L)] = jnp.minimum(acc, E - 1)
    cv = jnp.zeros((L,), jnp.int32)
    for e in range(E):
        cv = jnp.where(iota == e, counts[e], cv)
    cnt_v[...] = cv

    @pl.when(jnp.logical_and(cid == 0, sid == 0))
    def _():
        pltpu.sync_copy(be_v, be_hbm)
        pltpu.sync_copy(cnt_v, cnt_hbm)

    cur = list(start)
    for i in range(NV):
        v = e_chunk[pl.ds(i * L, L)]
        p = jnp.zeros((L,), jnp.int32)
        for e in range(E):
            m = v == e
            mi = m.astype(jnp.int32)
            cs = plsc.cumsum(mi)
            p = jnp.where(m, cur[e] + cs - 1, p)
            cur[e] = cur[e] + jnp.sum(mi)
        posv[pl.ds(i * L, L)] = p
        pos4[i // 8, pl.ds((i % 8) * L, L)] = p

    @pl.when(cid == 0)
    def _():
        pltpu.sync_copy(posv, pos_hbm.at[pl.ds(base, CHUNK)])
        for r in range(4):
            pltpu.async_copy(w4.at[r], gw_hbm.at[pos4.at[r]], sem).wait()

    for c4 in range(4):
        for jv in range(4):
            idx_e = c4 * 128 + 2 * (jv * L + iota)
            pe[c4, pl.ds(jv * L, L)] = plsc.load_gather(posv, [idx_e])
            po[c4, pl.ds(jv * L, L)] = plsc.load_gather(posv, [idx_e + 1])

    for c4 in range(4):
        tok0 = sid * TOKC + c4 * 64
        pltpu.sync_copy(x_hbm.at[pl.ds(tok0, 64)], xrows)
        if NC == 1:
            pltpu.async_copy(xrows, xg_hbm.at[pe.at[c4]], sem).wait()
            pltpu.async_copy(xrows, xg_hbm.at[po.at[c4]], sem).wait()
        else:
            @pl.when(cid == 0)
            def _():
                pltpu.async_copy(xrows, xg_hbm.at[pe.at[c4]], sem).wait()

            @pl.when(cid == 1)
            def _():
                pltpu.async_copy(xrows, xg_hbm.at[po.at[c4]], sem).wait()


def _mlp_body(be_ref, xg_ref, gw_ref, w1_ref, w2_ref, out_ref):
    f = pl.program_id(1)
    h = jax.nn.relu(
        lax.dot_general(xg_ref[...], w1_ref[0],
                        dimension_numbers=(((1,), (0,)), ((), ())),
                        preferred_element_type=jnp.float32))
    contrib = lax.dot_general(h, w2_ref[0],
                              dimension_numbers=(((1,), (0,)), ((), ())),
                              preferred_element_type=jnp.float32)

    @pl.when(f == 0)
    def _():
        out_ref[...] = contrib

    @pl.when(f > 0)
    def _():
        out_ref[...] = out_ref[...] + contrib

    @pl.when(f == NF - 1)
    def _():
        out_ref[...] = out_ref[...] * gw_ref[...]


_mlp = pl.pallas_call(
    _mlp_body,
    grid_spec=pltpu.PrefetchScalarGridSpec(
        num_scalar_prefetch=1,
        grid=(NB, NF),
        in_specs=[
            pl.BlockSpec((BLK, D), lambda b, f, be: (b, 0)),
            pl.BlockSpec((BLK, 1), lambda b, f, be: (b, 0)),
            pl.BlockSpec((1, D, FT), lambda b, f, be: (be[b], 0, f)),
            pl.BlockSpec((1, FT, D), lambda b, f, be: (be[b], f, 0)),
        ],
        out_specs=pl.BlockSpec((BLK, D), lambda b, f, be: (b, 0)),
    ),
    out_shape=jax.ShapeDtypeStruct((P, D), jnp.float32),
    compiler_params=pltpu.CompilerParams(
        dimension_semantics=("arbitrary", "arbitrary")),
)


@functools.partial(
    pl.kernel,
    mesh=_mesh,
    compiler_params=pltpu.CompilerParams(needs_layout_passes=False),
    out_type=jax.ShapeDtypeStruct((T, D), jnp.float32),
    scratch_types=[
        pltpu.VMEM((2 * TPW,), jnp.int32),
        pltpu.VMEM((4, 32), jnp.int32),
        pltpu.VMEM((4, 32), jnp.int32),
        pltpu.VMEM((32, D), jnp.float32),
        pltpu.VMEM((32, D), jnp.float32),
        pltpu.SemaphoreType.DMA,
    ],
)
def _combine(pos_hbm, yg_hbm, out_hbm, pidx, pe, po, buf0, buf1, sem):
    cid = lax.axis_index("c")
    sid = lax.axis_index("s")
    wid = sid * NC + cid
    tok0 = wid * TPW
    iota = lax.iota(jnp.int32, L)

    pltpu.sync_copy(pos_hbm.at[pl.ds(2 * tok0, 2 * TPW)], pidx)

    for c in range(TPW // 32):
        for jv in range(2):
            idx_e = c * 64 + 2 * (jv * L + iota)
            pe[c, pl.ds(jv * L, L)] = plsc.load_gather(pidx, [idx_e])
            po[c, pl.ds(jv * L, L)] = plsc.load_gather(pidx, [idx_e + 1])

    for c in range(TPW // 32):
        pltpu.async_copy(yg_hbm.at[pe.at[c]], buf0, sem).wait()
        pltpu.async_copy(yg_hbm.at[po.at[c]], buf1, sem).wait()

        def body(r, _):
            for k in range(D // L):
                a = buf0[r, pl.ds(k * L, L)]
                b = buf1[r, pl.ds(k * L, L)]
                buf0[r, pl.ds(k * L, L)] = a + b
            return 0

        lax.fori_loop(0, 32, body, 0)
        pltpu.sync_copy(buf0, out_hbm.at[pl.ds(tok0 + c * 32, 32)])


def kernel(x, expert_weights, W1, W2, top_experts):
    fe = top_experts.astype(jnp.int32).reshape(TK)
    ew = expert_weights.astype(jnp.float32).reshape(TK // 128, 128)
    xg, gw, be, pos, cnt = _route(x, fe, ew)
    yg = _mlp(be[:NB], xg, gw.reshape(P, 1), W1, W2)
    out = _combine(pos, yg)
    return (out, cnt[:E])


# speedup vs baseline: 2.7623x; 1.0243x over previous
import functools

import jax
import jax.numpy as jnp
from jax import lax
from jax.experimental import pallas as pl
from jax.experimental.pallas import tpu as pltpu
from jax.experimental.pallas import tpu_sc as plsc

E = 8
T = 4096
K = 2
TK = T * K
D = 1024
F = 4096
BLK = 256
NB = (TK + E * (BLK - 1) + BLK - 1) // BLK
P = NB * BLK
FT = 1024
NF = F // FT
BE_PAD = 64

_info = plsc.get_sparse_core_info()
NC = _info.num_cores
NS = _info.num_subcores
L = _info.num_lanes

CHUNK = TK // NS
NV = CHUNK // L
TOKC = T // NS
NW = NC * NS
TPW = T // NW

_mesh = plsc.VectorSubcoreMesh(core_axis_name="c", subcore_axis_name="s")


@functools.partial(
    pl.kernel,
    mesh=_mesh,
    compiler_params=pltpu.CompilerParams(needs_layout_passes=False),
    out_type=[
        jax.ShapeDtypeStruct((P, D), jnp.float32),
        jax.ShapeDtypeStruct((P,), jnp.float32),
        jax.ShapeDtypeStruct((BE_PAD,), jnp.int32),
        jax.ShapeDtypeStruct((TK,), jnp.int32),
        jax.ShapeDtypeStruct((L,), jnp.int32),
    ],
    scratch_types=[
        pltpu.VMEM((CHUNK,), jnp.int32),
        pltpu.VMEM((4, 128), jnp.float32),
        pltpu.VMEM((CHUNK,), jnp.int32),
        pltpu.VMEM((4, 128), jnp.int32),
        pltpu.VMEM((4, 64), jnp.int32),
        pltpu.VMEM((4, 64), jnp.int32),
        pltpu.VMEM((64, D), jnp.float32),
        pltpu.VMEM((L,), jnp.int32),
        pltpu.VMEM((NS, L), jnp.int32),
        pltpu.VMEM_SHARED((NS, L), jnp.int32),
        pltpu.VMEM((BE_PAD,), jnp.int32),
        pltpu.VMEM((L,), jnp.int32),
        pltpu.SemaphoreType.DMA,
    ],
)
def _route(x_hbm, fe_hbm, ew_hbm, xg_hbm, gw_hbm, be_hbm, pos_hbm, cnt_hbm,
           e_chunk, w4, posv, pos4, pe, po, xrows, hist_my, hist_all,
           hist_stage, be_v, cnt_v, sem):
    cid = lax.axis_index("c")
    sid = lax.axis_index("s")
    base = sid * CHUNK
    iota = lax.iota(jnp.int32, L)

    pltpu.sync_copy(fe_hbm.at[pl.ds(base, CHUNK)], e_chunk)
    pltpu.sync_copy(ew_hbm.at[pl.ds(sid * 4, 4)], w4)

    c_acc = [jnp.int32(0) for _ in range(E)]
    for i in range(NV):
        v = e_chunk[pl.ds(i * L, L)]
        for e in range(E):
            c_acc[e] = c_acc[e] + jnp.sum((v == e).astype(jnp.int32))
    hv = jnp.zeros((L,), jnp.int32)
    for e in range(E):
        hv = jnp.where(iota == e, c_acc[e], hv)
    hist_my[...] = hv
    pltpu.sync_copy(hist_my, hist_stage.at[sid])
    plsc.subcore_barrier()
    pltpu.sync_copy(hist_stage, hist_all)

    hrow = [hist_all[s, pl.ds(0, L)] for s in range(NS)]
    counts = []
    pref = []
    for e in range(E):
        tot = hrow[0][e]
        for s in range(1, NS):
            tot = tot + hrow[s][e]
        counts.append(tot)
        pr = jnp.int32(0)
        for s in range(NS):
            pr = pr + jnp.where(jnp.int32(s) < sid, hrow[s][e], 0)
        pref.append(pr)
    padded = [((counts[e] + (BLK - 1)) >> 8) << 8 for e in range(E)]
    excl = []
    cum = []
    run = jnp.int32(0)
    for e in range(E):
        excl.append(run)
        run = run + padded[e]
        cum.append(run)
    start = [excl[e] + pref[e] for e in range(E)]

    for j in range(BE_PAD // L):
        bv = (iota + j * L) * BLK
        acc = jnp.zeros((L,), jnp.int32)
        for e in range(E):
            acc = acc + (cum[e] <= bv).astype(jnp.int32)
        be_v[pl.ds(j * L, L)] = jnp.minimum(acc, E - 1)
    cv = jnp.zeros((L,), jnp.int32)
    for e in range(E):
        cv = jnp.where(iota == e, counts[e], cv)
    cnt_v[...] = cv

    @pl.when(jnp.logical_and(cid == 0, sid == 0))
    def _():
        pltpu.sync_copy(be_v, be_hbm)
        pltpu.sync_copy(cnt_v, cnt_hbm)

    cur = list(start)
    for i in range(NV):
        v = e_chunk[pl.ds(i * L, L)]
        p = jnp.zeros((L,), jnp.int32)
        for e in range(E):
            m = v == e
            mi = m.astype(jnp.int32)
            cs = plsc.cumsum(mi)
            p = jnp.where(m, cur[e] + cs - 1, p)
            cur[e] = cur[e] + jnp.sum(mi)
        posv[pl.ds(i * L, L)] = p
        pos4[i // 8, pl.ds((i % 8) * L, L)] = p

    @pl.when(cid == 0)
    def _():
        pltpu.sync_copy(posv, pos_hbm.at[pl.ds(base, CHUNK)])
        for r in range(4):
            pltpu.async_copy(w4.at[r], gw_hbm.at[pos4.at[r]], sem).wait()

    for c4 in range(4):
        for jv in range(4):
            idx_e = c4 * 128 + 2 * (jv * L + iota)
            pe[c4, pl.ds(jv * L, L)] = plsc.load_gather(posv, [idx_e])
            po[c4, pl.ds(jv * L, L)] = plsc.load_gather(posv, [idx_e + 1])

    for c4 in range(4):
        tok0 = sid * TOKC + c4 * 64
        pltpu.sync_copy(x_hbm.at[pl.ds(tok0, 64)], xrows)
        if NC == 1:
            pltpu.async_copy(xrows, xg_hbm.at[pe.at[c4]], sem).wait()
            pltpu.async_copy(xrows, xg_hbm.at[po.at[c4]], sem).wait()
        else:
            @pl.when(cid == 0)
            def _():
                pltpu.async_copy(xrows, xg_hbm.at[pe.at[c4]], sem).wait()

            @pl.when(cid == 1)
            def _():
                pltpu.async_copy(xrows, xg_hbm.at[po.at[c4]], sem).wait()


def _mlp_body(be_ref, xg_ref, gw_ref, w1_ref, w2_ref, out_ref):
    f = pl.program_id(1)
    xb = xg_ref[...].astype(jnp.bfloat16)
    h = jax.nn.relu(
        lax.dot_general(xb, w1_ref[0],
                        dimension_numbers=(((1,), (0,)), ((), ())),
                        preferred_element_type=jnp.float32))
    contrib = lax.dot_general(h.astype(jnp.bfloat16), w2_ref[0],
                              dimension_numbers=(((1,), (0,)), ((), ())),
                              preferred_element_type=jnp.float32)

    @pl.when(f == 0)
    def _():
        out_ref[...] = contrib

    @pl.when(f > 0)
    def _():
        out_ref[...] = out_ref[...] + contrib

    @pl.when(f == NF - 1)
    def _():
        out_ref[...] = out_ref[...] * gw_ref[...]


_mlp = pl.pallas_call(
    _mlp_body,
    grid_spec=pltpu.PrefetchScalarGridSpec(
        num_scalar_prefetch=1,
        grid=(NB, NF),
        in_specs=[
            pl.BlockSpec((BLK, D), lambda b, f, be: (b, 0)),
            pl.BlockSpec((BLK, 1), lambda b, f, be: (b, 0)),
            pl.BlockSpec((1, D, FT), lambda b, f, be: (be[b], 0, f)),
            pl.BlockSpec((1, FT, D), lambda b, f, be: (be[b], f, 0)),
        ],
        out_specs=pl.BlockSpec((BLK, D), lambda b, f, be: (b, 0)),
    ),
    out_shape=jax.ShapeDtypeStruct((P, D), jnp.float32),
    compiler_params=pltpu.CompilerParams(
        dimension_semantics=("arbitrary", "arbitrary")),
)


@functools.partial(
    pl.kernel,
    mesh=_mesh,
    compiler_params=pltpu.CompilerParams(needs_layout_passes=False),
    out_type=jax.ShapeDtypeStruct((T, D), jnp.float32),
    scratch_types=[
        pltpu.VMEM((2 * TPW,), jnp.int32),
        pltpu.VMEM((4, 32), jnp.int32),
        pltpu.VMEM((4, 32), jnp.int32),
        pltpu.VMEM((32, D), jnp.float32),
        pltpu.VMEM((32, D), jnp.float32),
        pltpu.SemaphoreType.DMA,
    ],
)
def _combine(pos_hbm, yg_hbm, out_hbm, pidx, pe, po, buf0, buf1, sem):
    cid = lax.axis_index("c")
    sid = lax.axis_index("s")
    wid = sid * NC + cid
    tok0 = wid * TPW
    iota = lax.iota(jnp.int32, L)

    pltpu.sync_copy(pos_hbm.at[pl.ds(2 * tok0, 2 * TPW)], pidx)

    for c in range(TPW // 32):
        for jv in range(2):
            idx_e = c * 64 + 2 * (jv * L + iota)
            pe[c, pl.ds(jv * L, L)] = plsc.load_gather(pidx, [idx_e])
            po[c, pl.ds(jv * L, L)] = plsc.load_gather(pidx, [idx_e + 1])

    for c in range(TPW // 32):
        pltpu.async_copy(yg_hbm.at[pe.at[c]], buf0, sem).wait()
        pltpu.async_copy(yg_hbm.at[po.at[c]], buf1, sem).wait()

        def body(r, _):
            for k in range(D // L):
                a = buf0[r, pl.ds(k * L, L)]
                b = buf1[r, pl.ds(k * L, L)]
                buf0[r, pl.ds(k * L, L)] = a + b
            return 0

        lax.fori_loop(0, 32, body, 0)
        pltpu.sync_copy(buf0, out_hbm.at[pl.ds(tok0 + c * 32, 32)])


def kernel(x, expert_weights, W1, W2, top_experts):
    fe = top_experts.astype(jnp.int32).reshape(TK)
    ew = expert_weights.astype(jnp.float32).reshape(TK // 128, 128)
    xg, gw, be, pos, cnt = _route(x, fe, ew)
    yg = _mlp(be[:NB], xg, gw.reshape(P, 1),
              W1.astype(jnp.bfloat16), W2.astype(jnp.bfloat16))
    out = _combine(pos, yg)
    return (out, cnt[:E])


# speedup vs baseline: 3.4865x; 1.2622x over previous
import functools

import jax
import jax.numpy as jnp
from jax import lax
from jax.experimental import pallas as pl
from jax.experimental.pallas import tpu as pltpu
from jax.experimental.pallas import tpu_sc as plsc

E = 8
T = 4096
K = 2
TK = T * K
D = 1024
F = 4096
BLK = 256
NB = (TK + E * (BLK - 1) + BLK - 1) // BLK
P = NB * BLK
FT = 1024
NF = F // FT
BE_PAD = 64

_info = plsc.get_sparse_core_info()
NC = _info.num_cores
NS = _info.num_subcores
L = _info.num_lanes

CHUNK = TK // NS
NV = CHUNK // L
TOKC = T // NS
NW = NC * NS
TPW = T // NW

_mesh = plsc.VectorSubcoreMesh(core_axis_name="c", subcore_axis_name="s")


@functools.partial(
    pl.kernel,
    mesh=_mesh,
    compiler_params=pltpu.CompilerParams(needs_layout_passes=False),
    out_type=[
        jax.ShapeDtypeStruct((P, D), jnp.float32),
        jax.ShapeDtypeStruct((P,), jnp.float32),
        jax.ShapeDtypeStruct((BE_PAD,), jnp.int32),
        jax.ShapeDtypeStruct((TK,), jnp.int32),
        jax.ShapeDtypeStruct((L,), jnp.int32),
    ],
    scratch_types=[
        pltpu.VMEM((CHUNK,), jnp.int32),
        pltpu.VMEM((4, 128), jnp.float32),
        pltpu.VMEM((CHUNK,), jnp.int32),
        pltpu.VMEM((4, 128), jnp.int32),
        pltpu.VMEM((4, 64), jnp.int32),
        pltpu.VMEM((4, 64), jnp.int32),
        pltpu.VMEM((64, D), jnp.float32),
        pltpu.VMEM((L,), jnp.int32),
        pltpu.VMEM((NS, L), jnp.int32),
        pltpu.VMEM_SHARED((NS, L), jnp.int32),
        pltpu.VMEM((BE_PAD,), jnp.int32),
        pltpu.VMEM((L,), jnp.int32),
        pltpu.SemaphoreType.DMA,
    ],
)
def _route(x_hbm, fe_hbm, ew_hbm, xg_hbm, gw_hbm, be_hbm, pos_hbm, cnt_hbm,
           e_chunk, w4, posv, pos4, pe, po, xrows, hist_my, hist_all,
           hist_stage, be_v, cnt_v, sem):
    cid = lax.axis_index("c")
    sid = lax.axis_index("s")
    base = sid * CHUNK
    iota = lax.iota(jnp.int32, L)

    pltpu.sync_copy(fe_hbm.at[pl.ds(base, CHUNK)], e_chunk)
    pltpu.sync_copy(ew_hbm.at[pl.ds(sid * 4, 4)], w4)

    c_acc = [jnp.int32(0) for _ in range(E)]
    for i in range(NV):
        v = e_chunk[pl.ds(i * L, L)]
        for e in range(E):
            c_acc[e] = c_acc[e] + jnp.sum((v == e).astype(jnp.int32))
    hv = jnp.zeros((L,), jnp.int32)
    for e in range(E):
        hv = jnp.where(iota == e, c_acc[e], hv)
    hist_my[...] = hv
    pltpu.sync_copy(hist_my, hist_stage.at[sid])
    plsc.subcore_barrier()
    pltpu.sync_copy(hist_stage, hist_all)

    hrow = [hist_all[s, pl.ds(0, L)] for s in range(NS)]
    counts = []
    pref = []
    for e in range(E):
        tot = hrow[0][e]
        for s in range(1, NS):
            tot = tot + hrow[s][e]
        counts.append(tot)
        pr = jnp.int32(0)
        for s in range(NS):
            pr = pr + jnp.where(jnp.int32(s) < sid, hrow[s][e], 0)
        pref.append(pr)
    padded = [((counts[e] + (BLK - 1)) >> 8) << 8 for e in range(E)]
    excl = []
    cum = []
    run = jnp.int32(0)
    for e in range(E):
        excl.append(run)
        run = run + padded[e]
        cum.append(run)
    start = [excl[e] + pref[e] for e in range(E)]

    for j in range(BE_PAD // L):
        bv = (iota + j * L) * BLK
        acc = jnp.zeros((L,), jnp.int32)
        for e in range(E):
            acc = acc + (cum[e] <= bv).astype(jnp.int32)
        be_v[pl.ds(j * L, L)] = jnp.minimum(acc, E - 1)
    cv = jnp.zeros((L,), jnp.int32)
    for e in range(E):
        cv = jnp.where(iota == e, counts[e], cv)
    cnt_v[...] = cv

    @pl.when(jnp.logical_and(cid == 0, sid == 0))
    def _():
        pltpu.sync_copy(be_v, be_hbm)
        pltpu.sync_copy(cnt_v, cnt_hbm)

    cur = list(start)
    for i in range(NV):
        v = e_chunk[pl.ds(i * L, L)]
        p = jnp.zeros((L,), jnp.int32)
        for e in range(E):
            m = v == e
            mi = m.astype(jnp.int32)
            cs = plsc.cumsum(mi)
            p = jnp.where(m, cur[e] + cs - 1, p)
            cur[e] = cur[e] + jnp.sum(mi)
        posv[pl.ds(i * L, L)] = p
        pos4[i // 8, pl.ds((i % 8) * L, L)] = p

    @pl.when(cid == 0)
    def _():
        pltpu.sync_copy(posv, pos_hbm.at[pl.ds(base, CHUNK)])
        for r in range(4):
            pltpu.async_copy(w4.at[r], gw_hbm.at[pos4.at[r]], sem).wait()

    for c4 in range(4):
        for jv in range(4):
            idx_e = c4 * 128 + 2 * (jv * L + iota)
            pe[c4, pl.ds(jv * L, L)] = plsc.load_gather(posv, [idx_e])
            po[c4, pl.ds(jv * L, L)] = plsc.load_gather(posv, [idx_e + 1])

    for c4 in range(4):
        tok0 = sid * TOKC + c4 * 64
        pltpu.sync_copy(x_hbm.at[pl.ds(tok0, 64)], xrows)
        if NC == 1:
            pltpu.async_copy(xrows, xg_hbm.at[pe.at[c4]], sem).wait()
            pltpu.async_copy(xrows, xg_hbm.at[po.at[c4]], sem).wait()
        else:
            @pl.when(cid == 0)
            def _():
                pltpu.async_copy(xrows, xg_hbm.at[pe.at[c4]], sem).wait()

            @pl.when(cid == 1)
            def _():
                pltpu.async_copy(xrows, xg_hbm.at[po.at[c4]], sem).wait()


def _mlp_body(be_ref, xg_ref, gw_ref, w1_ref, w2_ref, out_ref):
    xb = xg_ref[...].astype(jnp.bfloat16)
    acc = jnp.zeros((BLK, D), jnp.float32)
    for f in range(NF):
        h = jax.nn.relu(
            lax.dot_general(xb, w1_ref[0, :, f * FT:(f + 1) * FT],
                            dimension_numbers=(((1,), (0,)), ((), ())),
                            preferred_element_type=jnp.float32))
        acc = acc + lax.dot_general(h.astype(jnp.bfloat16),
                                    w2_ref[0, f * FT:(f + 1) * FT, :],
                                    dimension_numbers=(((1,), (0,)), ((), ())),
                                    preferred_element_type=jnp.float32)
    out_ref[...] = acc * gw_ref[...]


_mlp = pl.pallas_call(
    _mlp_body,
    grid_spec=pltpu.PrefetchScalarGridSpec(
        num_scalar_prefetch=1,
        grid=(NB,),
        in_specs=[
            pl.BlockSpec((BLK, D), lambda b, be: (b, 0)),
            pl.BlockSpec((BLK, 1), lambda b, be: (b, 0)),
            pl.BlockSpec((1, D, F), lambda b, be: (be[b], 0, 0)),
            pl.BlockSpec((1, F, D), lambda b, be: (be[b], 0, 0)),
        ],
        out_specs=pl.BlockSpec((BLK, D), lambda b, be: (b, 0)),
    ),
    out_shape=jax.ShapeDtypeStruct((P, D), jnp.float32),
    compiler_params=pltpu.CompilerParams(
        dimension_semantics=("arbitrary",)),
)


@functools.partial(
    pl.kernel,
    mesh=_mesh,
    compiler_params=pltpu.CompilerParams(needs_layout_passes=False),
    out_type=jax.ShapeDtypeStruct((T, D), jnp.float32),
    scratch_types=[
        pltpu.VMEM((2 * TPW,), jnp.int32),
        pltpu.VMEM((4, 32), jnp.int32),
        pltpu.VMEM((4, 32), jnp.int32),
        pltpu.VMEM((32, D), jnp.float32),
        pltpu.VMEM((32, D), jnp.float32),
        pltpu.SemaphoreType.DMA,
    ],
)
def _combine(pos_hbm, yg_hbm, out_hbm, pidx, pe, po, buf0, buf1, sem):
    cid = lax.axis_index("c")
    sid = lax.axis_index("s")
    wid = sid * NC + cid
    tok0 = wid * TPW
    iota = lax.iota(jnp.int32, L)

    pltpu.sync_copy(pos_hbm.at[pl.ds(2 * tok0, 2 * TPW)], pidx)

    for c in range(TPW // 32):
        for jv in range(2):
            idx_e = c * 64 + 2 * (jv * L + iota)
            pe[c, pl.ds(jv * L, L)] = plsc.load_gather(pidx, [idx_e])
            po[c, pl.ds(jv * L, L)] = plsc.load_gather(pidx, [idx_e + 1])

    for c in range(TPW // 32):
        pltpu.async_copy(yg_hbm.at[pe.at[c]], buf0, sem).wait()
        pltpu.async_copy(yg_hbm.at[po.at[c]], buf1, sem).wait()

        def body(r, _):
            for k in range(D // L):
                a = buf0[r, pl.ds(k * L, L)]
                b = buf1[r, pl.ds(k * L, L)]
                buf0[r, pl.ds(k * L, L)] = a + b
            return 0

        lax.fori_loop(0, 32, body, 0)
        pltpu.sync_copy(buf0, out_hbm.at[pl.ds(tok0 + c * 32, 32)])


def kernel(x, expert_weights, W1, W2, top_experts):
    fe = top_experts.astype(jnp.int32).reshape(TK)
    ew = expert_weights.astype(jnp.float32).reshape(TK // 128, 128)
    xg, gw, be, pos, cnt = _route(x, fe, ew)
    yg = _mlp(be[:NB], xg, gw.reshape(P, 1),
              W1.astype(jnp.bfloat16), W2.astype(jnp.bfloat16))
    out = _combine(pos, yg)
    return (out, cnt[:E])


# speedup vs baseline: 3.5847x; 1.0282x over previous
import functools

import jax
import jax.numpy as jnp
from jax import lax
from jax.experimental import pallas as pl
from jax.experimental.pallas import tpu as pltpu
from jax.experimental.pallas import tpu_sc as plsc

E = 8
T = 4096
K = 2
TK = T * K
D = 1024
F = 4096
BLK = 256
NB = (TK + E * (BLK - 1) + BLK - 1) // BLK
P = NB * BLK
FT = 1024
NF = F // FT
BE_PAD = 64

_info = plsc.get_sparse_core_info()
NC = _info.num_cores
NS = _info.num_subcores
L = _info.num_lanes

CHUNK = TK // NS
NV = CHUNK // L
TOKC = T // NS
NW = NC * NS
TPW = T // NW

_mesh = plsc.VectorSubcoreMesh(core_axis_name="c", subcore_axis_name="s")


@functools.partial(
    pl.kernel,
    mesh=_mesh,
    compiler_params=pltpu.CompilerParams(needs_layout_passes=False),
    out_type=[
        jax.ShapeDtypeStruct((P, D), jnp.float32),
        jax.ShapeDtypeStruct((BE_PAD,), jnp.int32),
        jax.ShapeDtypeStruct((TK,), jnp.int32),
        jax.ShapeDtypeStruct((L,), jnp.int32),
    ],
    scratch_types=[
        pltpu.VMEM((CHUNK,), jnp.int32),
        pltpu.VMEM((CHUNK,), jnp.int32),
        pltpu.VMEM((4, 64), jnp.int32),
        pltpu.VMEM((4, 64), jnp.int32),
        pltpu.VMEM((64, D), jnp.float32),
        pltpu.VMEM((L,), jnp.int32),
        pltpu.VMEM((NS, L), jnp.int32),
        pltpu.VMEM_SHARED((NS, L), jnp.int32),
        pltpu.VMEM((BE_PAD,), jnp.int32),
        pltpu.VMEM((L,), jnp.int32),
        pltpu.SemaphoreType.DMA,
    ],
)
def _route(x_hbm, fe_hbm, xg_hbm, be_hbm, pos_hbm, cnt_hbm,
           e_chunk, posv, pe, po, xrows, hist_my, hist_all,
           hist_stage, be_v, cnt_v, sem):
    cid = lax.axis_index("c")
    sid = lax.axis_index("s")
    base = sid * CHUNK
    iota = lax.iota(jnp.int32, L)

    pltpu.sync_copy(fe_hbm.at[pl.ds(base, CHUNK)], e_chunk)

    c_acc = [jnp.int32(0) for _ in range(E)]
    for i in range(NV):
        v = e_chunk[pl.ds(i * L, L)]
        for e in range(E):
            c_acc[e] = c_acc[e] + jnp.sum((v == e).astype(jnp.int32))
    hv = jnp.zeros((L,), jnp.int32)
    for e in range(E):
        hv = jnp.where(iota == e, c_acc[e], hv)
    hist_my[...] = hv
    pltpu.sync_copy(hist_my, hist_stage.at[sid])
    plsc.subcore_barrier()
    pltpu.sync_copy(hist_stage, hist_all)

    hrow = [hist_all[s, pl.ds(0, L)] for s in range(NS)]
    counts = []
    pref = []
    for e in range(E):
        tot = hrow[0][e]
        for s in range(1, NS):
            tot = tot + hrow[s][e]
        counts.append(tot)
        pr = jnp.int32(0)
        for s in range(NS):
            pr = pr + jnp.where(jnp.int32(s) < sid, hrow[s][e], 0)
        pref.append(pr)
    padded = [((counts[e] + (BLK - 1)) >> 8) << 8 for e in range(E)]
    excl = []
    cum = []
    run = jnp.int32(0)
    for e in range(E):
        excl.append(run)
        run = run + padded[e]
        cum.append(run)
    start = [excl[e] + pref[e] for e in range(E)]

    for j in range(BE_PAD // L):
        bv = (iota + j * L) * BLK
        acc = jnp.zeros((L,), jnp.int32)
        for e in range(E):
            acc = acc + (cum[e] <= bv).astype(jnp.int32)
        be_v[pl.ds(j * L, L)] = jnp.minimum(acc, E - 1)
    cv = jnp.zeros((L,), jnp.int32)
    for e in range(E):
        cv = jnp.where(iota == e, counts[e], cv)
    cnt_v[...] = cv

    @pl.when(jnp.logical_and(cid == 0, sid == 0))
    def _():
        pltpu.sync_copy(be_v, be_hbm)
        pltpu.sync_copy(cnt_v, cnt_hbm)

    cur = list(start)
    for i in range(NV):
        v = e_chunk[pl.ds(i * L, L)]
        p = jnp.zeros((L,), jnp.int32)
        for e in range(E):
            m = v == e
            mi = m.astype(jnp.int32)
            cs = plsc.cumsum(mi)
            p = jnp.where(m, cur[e] + cs - 1, p)
            cur[e] = cur[e] + jnp.sum(mi)
        posv[pl.ds(i * L, L)] = p

    @pl.when(cid == 0)
    def _():
        pltpu.sync_copy(posv, pos_hbm.at[pl.ds(base, CHUNK)])

    for c4 in range(4):
        for jv in range(4):
            idx_e = c4 * 128 + 2 * (jv * L + iota)
            pe[c4, pl.ds(jv * L, L)] = plsc.load_gather(posv, [idx_e])
            po[c4, pl.ds(jv * L, L)] = plsc.load_gather(posv, [idx_e + 1])

    for c4 in range(4):
        tok0 = sid * TOKC + c4 * 64
        pltpu.sync_copy(x_hbm.at[pl.ds(tok0, 64)], xrows)
        if NC == 1:
            pltpu.async_copy(xrows, xg_hbm.at[pe.at[c4]], sem).wait()
            pltpu.async_copy(xrows, xg_hbm.at[po.at[c4]], sem).wait()
        else:
            @pl.when(cid == 0)
            def _():
                pltpu.async_copy(xrows, xg_hbm.at[pe.at[c4]], sem).wait()

            @pl.when(cid == 1)
            def _():
                pltpu.async_copy(xrows, xg_hbm.at[po.at[c4]], sem).wait()


def _mlp_body(be_ref, xg_ref, w1_ref, w2_ref, out_ref):
    xb = xg_ref[...].astype(jnp.bfloat16)
    acc = jnp.zeros((BLK, D), jnp.float32)
    for f in range(NF):
        h = jax.nn.relu(
            lax.dot_general(xb, w1_ref[0, :, f * FT:(f + 1) * FT],
                            dimension_numbers=(((1,), (0,)), ((), ())),
                            preferred_element_type=jnp.float32))
        acc = acc + lax.dot_general(h.astype(jnp.bfloat16),
                                    w2_ref[0, f * FT:(f + 1) * FT, :],
                                    dimension_numbers=(((1,), (0,)), ((), ())),
                                    preferred_element_type=jnp.float32)
    out_ref[...] = acc


_mlp = pl.pallas_call(
    _mlp_body,
    grid_spec=pltpu.PrefetchScalarGridSpec(
        num_scalar_prefetch=1,
        grid=(NB,),
        in_specs=[
            pl.BlockSpec((BLK, D), lambda b, be: (b, 0)),
            pl.BlockSpec((1, D, F), lambda b, be: (be[b], 0, 0)),
            pl.BlockSpec((1, F, D), lambda b, be: (be[b], 0, 0)),
        ],
        out_specs=pl.BlockSpec((BLK, D), lambda b, be: (b, 0)),
    ),
    out_shape=jax.ShapeDtypeStruct((P, D), jnp.float32),
    compiler_params=pltpu.CompilerParams(
        dimension_semantics=("arbitrary",)),
)


@functools.partial(
    pl.kernel,
    mesh=_mesh,
    compiler_params=pltpu.CompilerParams(needs_layout_passes=False),
    out_type=jax.ShapeDtypeStruct((T, D), jnp.float32),
    scratch_types=[
        pltpu.VMEM((2 * TPW,), jnp.int32),
        pltpu.VMEM((2 * TPW,), jnp.float32),
        pltpu.VMEM((4, 32), jnp.int32),
        pltpu.VMEM((4, 32), jnp.int32),
        pltpu.VMEM((32, D), jnp.float32),
        pltpu.VMEM((32, D), jnp.float32),
        pltpu.SemaphoreType.DMA,
    ],
)
def _combine(ew_hbm, pos_hbm, yg_hbm, out_hbm, pidx, ewv, pe, po, buf0, buf1, sem):
    cid = lax.axis_index("c")
    sid = lax.axis_index("s")
    wid = sid * NC + cid
    tok0 = wid * TPW
    iota = lax.iota(jnp.int32, L)

    pltpu.sync_copy(pos_hbm.at[pl.ds(2 * tok0, 2 * TPW)], pidx)
    pltpu.sync_copy(ew_hbm.at[pl.ds(2 * tok0, 2 * TPW)], ewv)

    for c in range(TPW // 32):
        for jv in range(2):
            idx_e = c * 64 + 2 * (jv * L + iota)
            pe[c, pl.ds(jv * L, L)] = plsc.load_gather(pidx, [idx_e])
            po[c, pl.ds(jv * L, L)] = plsc.load_gather(pidx, [idx_e + 1])

    for c in range(TPW // 32):
        pltpu.async_copy(yg_hbm.at[pe.at[c]], buf0, sem).wait()
        pltpu.async_copy(yg_hbm.at[po.at[c]], buf1, sem).wait()

        def body(r, _, c=c):
            g0 = jnp.full((L,), 2 * (c * 32 + r), jnp.int32)
            w0 = plsc.load_gather(ewv, [g0])
            w1 = plsc.load_gather(ewv, [g0 + 1])
            for k in range(D // L):
                a = buf0[r, pl.ds(k * L, L)]
                b = buf1[r, pl.ds(k * L, L)]
                buf0[r, pl.ds(k * L, L)] = a * w0 + b * w1
            return 0

        lax.fori_loop(0, 32, body, 0)
        pltpu.sync_copy(buf0, out_hbm.at[pl.ds(tok0 + c * 32, 32)])


def kernel(x, expert_weights, W1, W2, top_experts):
    fe = top_experts.astype(jnp.int32).reshape(TK)
    ew = expert_weights.astype(jnp.float32).reshape(TK)
    xg, be, pos, cnt = _route(x, fe)
    yg = _mlp(be[:NB], xg,
              W1.astype(jnp.bfloat16), W2.astype(jnp.bfloat16))
    out = _combine(ew, pos, yg)
    return (out, cnt[:E])


# speedup vs baseline: 3.9230x; 1.0944x over previous
import functools

import jax
import jax.numpy as jnp
from jax import lax
from jax.experimental import pallas as pl
from jax.experimental.pallas import tpu as pltpu
from jax.experimental.pallas import tpu_sc as plsc

E = 8
T = 4096
K = 2
TK = T * K
D = 1024
F = 4096
BLKLOG = 9
BLK = 1 << BLKLOG
NB = (TK + E * (BLK - 1) + BLK - 1) // BLK
P = NB * BLK
FH = F // 2
BE_PAD = 64

_info = plsc.get_sparse_core_info()
NC = _info.num_cores
NS = _info.num_subcores
L = _info.num_lanes

CHUNK = TK // NS
NV = CHUNK // L
TOKC = T // NS
NW = NC * NS
TPW = T // NW

_mesh = plsc.VectorSubcoreMesh(core_axis_name="c", subcore_axis_name="s")


@functools.partial(
    pl.kernel,
    mesh=_mesh,
    compiler_params=pltpu.CompilerParams(needs_layout_passes=False),
    out_type=[
        jax.ShapeDtypeStruct((P, D), jnp.float32),
        jax.ShapeDtypeStruct((BE_PAD,), jnp.int32),
        jax.ShapeDtypeStruct((TK,), jnp.int32),
        jax.ShapeDtypeStruct((L,), jnp.int32),
    ],
    scratch_types=[
        pltpu.VMEM((CHUNK,), jnp.int32),
        pltpu.VMEM((CHUNK,), jnp.int32),
        pltpu.VMEM((4, 64), jnp.int32),
        pltpu.VMEM((4, 64), jnp.int32),
        pltpu.VMEM((64, D), jnp.float32),
        pltpu.VMEM((L,), jnp.int32),
        pltpu.VMEM((NS, L), jnp.int32),
        pltpu.VMEM_SHARED((NS, L), jnp.int32),
        pltpu.VMEM((BE_PAD,), jnp.int32),
        pltpu.VMEM((L,), jnp.int32),
        pltpu.SemaphoreType.DMA,
    ],
)
def _route(x_hbm, fe_hbm, xg_hbm, be_hbm, pos_hbm, cnt_hbm,
           e_chunk, posv, pe, po, xrows, hist_my, hist_all,
           hist_stage, be_v, cnt_v, sem):
    cid = lax.axis_index("c")
    sid = lax.axis_index("s")
    base = sid * CHUNK
    iota = lax.iota(jnp.int32, L)

    pltpu.sync_copy(fe_hbm.at[pl.ds(base, CHUNK)], e_chunk)

    c_acc = [jnp.int32(0) for _ in range(E)]
    for i in range(NV):
        v = e_chunk[pl.ds(i * L, L)]
        for e in range(E):
            c_acc[e] = c_acc[e] + jnp.sum((v == e).astype(jnp.int32))
    hv = jnp.zeros((L,), jnp.int32)
    for e in range(E):
        hv = jnp.where(iota == e, c_acc[e], hv)
    hist_my[...] = hv
    pltpu.sync_copy(hist_my, hist_stage.at[sid])
    plsc.subcore_barrier()
    pltpu.sync_copy(hist_stage, hist_all)

    hrow = [hist_all[s, pl.ds(0, L)] for s in range(NS)]
    counts = []
    pref = []
    for e in range(E):
        tot = hrow[0][e]
        for s in range(1, NS):
            tot = tot + hrow[s][e]
        counts.append(tot)
        pr = jnp.int32(0)
        for s in range(NS):
            pr = pr + jnp.where(jnp.int32(s) < sid, hrow[s][e], 0)
        pref.append(pr)
    padded = [((counts[e] + (BLK - 1)) >> BLKLOG) << BLKLOG for e in range(E)]
    excl = []
    cum = []
    run = jnp.int32(0)
    for e in range(E):
        excl.append(run)
        run = run + padded[e]
        cum.append(run)
    start = [excl[e] + pref[e] for e in range(E)]

    for j in range(BE_PAD // L):
        bv = (iota + j * L) * BLK
        acc = jnp.zeros((L,), jnp.int32)
        for e in range(E):
            acc = acc + (cum[e] <= bv).astype(jnp.int32)
        bev = jnp.minimum(acc, E - 1)
        if j == BE_PAD // L - 1:
            bev = jnp.where(iota == L - 1, cum[E - 1] >> BLKLOG, bev)
        be_v[pl.ds(j * L, L)] = bev
    cv = jnp.zeros((L,), jnp.int32)
    for e in range(E):
        cv = jnp.where(iota == e, counts[e], cv)
    cnt_v[...] = cv

    @pl.when(jnp.logical_and(cid == 0, sid == 0))
    def _():
        pltpu.sync_copy(be_v, be_hbm)
        pltpu.sync_copy(cnt_v, cnt_hbm)

    cur = list(start)
    for i in range(NV):
        v = e_chunk[pl.ds(i * L, L)]
        p = jnp.zeros((L,), jnp.int32)
        for e in range(E):
            m = v == e
            mi = m.astype(jnp.int32)
            cs = plsc.cumsum(mi)
            p = jnp.where(m, cur[e] + cs - 1, p)
            cur[e] = cur[e] + jnp.sum(mi)
        posv[pl.ds(i * L, L)] = p

    @pl.when(cid == 0)
    def _():
        pltpu.sync_copy(posv, pos_hbm.at[pl.ds(base, CHUNK)])

    for c4 in range(4):
        for jv in range(4):
            idx_e = c4 * 128 + 2 * (jv * L + iota)
            pe[c4, pl.ds(jv * L, L)] = plsc.load_gather(posv, [idx_e])
            po[c4, pl.ds(jv * L, L)] = plsc.load_gather(posv, [idx_e + 1])

    for c4 in range(4):
        tok0 = sid * TOKC + c4 * 64
        pltpu.sync_copy(x_hbm.at[pl.ds(tok0, 64)], xrows)
        if NC == 1:
            pltpu.async_copy(xrows, xg_hbm.at[pe.at[c4]], sem).wait()
            pltpu.async_copy(xrows, xg_hbm.at[po.at[c4]], sem).wait()
        else:
            @pl.when(cid == 0)
            def _():
                pltpu.async_copy(xrows, xg_hbm.at[pe.at[c4]], sem).wait()

            @pl.when(cid == 1)
            def _():
                pltpu.async_copy(xrows, xg_hbm.at[po.at[c4]], sem).wait()


def _mlp_body(be_ref, xg_ref, yin_ref, w1_ref, w2_ref, out_ref):
    j = pl.program_id(0)
    b = pl.program_id(1)
    used = be_ref[BE_PAD - 1]

    @pl.when(b < used)
    def _():
        xb = xg_ref[...].astype(jnp.bfloat16)
        h = jax.nn.relu(
            lax.dot_general(xb, w1_ref[0],
                            dimension_numbers=(((1,), (0,)), ((), ())),
                            preferred_element_type=jnp.float32))
        contrib = lax.dot_general(h.astype(jnp.bfloat16), w2_ref[0],
                                  dimension_numbers=(((1,), (0,)), ((), ())),
                                  preferred_element_type=jnp.float32)

        @pl.when(j == 0)
        def _():
            out_ref[...] = contrib

        @pl.when(j > 0)
        def _():
            out_ref[...] = yin_ref[...] + contrib


_mlp = pl.pallas_call(
    _mlp_body,
    grid_spec=pltpu.PrefetchScalarGridSpec(
        num_scalar_prefetch=1,
        grid=(2, NB),
        in_specs=[
            pl.BlockSpec((BLK, D), lambda j, b, be: (b, 0)),
            pl.BlockSpec((BLK, D), lambda j, b, be: (b, 0)),
            pl.BlockSpec((1, D, FH), lambda j, b, be: (be[b], 0, j)),
            pl.BlockSpec((1, FH, D), lambda j, b, be: (be[b], j, 0)),
        ],
        out_specs=pl.BlockSpec((BLK, D), lambda j, b, be: (b, 0)),
    ),
    out_shape=jax.ShapeDtypeStruct((P, D), jnp.float32),
    input_output_aliases={2: 0},
    compiler_params=pltpu.CompilerParams(
        dimension_semantics=("arbitrary", "arbitrary")),
)


@functools.partial(
    pl.kernel,
    mesh=_mesh,
    compiler_params=pltpu.CompilerParams(needs_layout_passes=False),
    out_type=jax.ShapeDtypeStruct((T, D), jnp.float32),
    scratch_types=[
        pltpu.VMEM((2 * TPW,), jnp.int32),
        pltpu.VMEM((2 * TPW,), jnp.float32),
        pltpu.VMEM((4, 32), jnp.int32),
        pltpu.VMEM((4, 32), jnp.int32),
        pltpu.VMEM((32, D), jnp.float32),
        pltpu.VMEM((32, D), jnp.float32),
        pltpu.SemaphoreType.DMA,
    ],
)
def _combine(ew_hbm, pos_hbm, yg_hbm, out_hbm, pidx, ewv, pe, po,
             buf0, buf1, sem):
    cid = lax.axis_index("c")
    sid = lax.axis_index("s")
    wid = sid * NC + cid
    tok0 = wid * TPW
    iota = lax.iota(jnp.int32, L)

    pltpu.sync_copy(pos_hbm.at[pl.ds(2 * tok0, 2 * TPW)], pidx)
    pltpu.sync_copy(ew_hbm.at[pl.ds(2 * tok0, 2 * TPW)], ewv)

    for c in range(TPW // 32):
        for jv in range(2):
            idx_e = c * 64 + 2 * (jv * L + iota)
            pe[c, pl.ds(jv * L, L)] = plsc.load_gather(pidx, [idx_e])
            po[c, pl.ds(jv * L, L)] = plsc.load_gather(pidx, [idx_e + 1])

    for c in range(TPW // 32):
        pltpu.async_copy(yg_hbm.at[pe.at[c]], buf0, sem).wait()
        pltpu.async_copy(yg_hbm.at[po.at[c]], buf1, sem).wait()

        def body(r, _, c=c):
            g0 = jnp.full((L,), 2 * (c * 32 + r), jnp.int32)
            w0 = plsc.load_gather(ewv, [g0])
            w1 = plsc.load_gather(ewv, [g0 + 1])
            for k in range(D // L):
                a = buf0[r, pl.ds(k * L, L)]
                b = buf1[r, pl.ds(k * L, L)]
                buf0[r, pl.ds(k * L, L)] = a * w0 + b * w1
            return 0

        lax.fori_loop(0, 32, body, 0)
        pltpu.sync_copy(buf0, out_hbm.at[pl.ds(tok0 + c * 32, 32)])


def kernel(x, expert_weights, W1, W2, top_experts):
    fe = top_experts.astype(jnp.int32).reshape(TK)
    ew = expert_weights.astype(jnp.float32).reshape(TK)
    xg, be, pos, cnt = _route(x, fe)
    yg = _mlp(be, xg, jnp.zeros((P, D), jnp.float32), W1, W2)
    out = _combine(ew, pos, yg)
    return (out, cnt[:E])


# speedup vs baseline: 4.1650x; 1.0617x over previous
import functools

import jax
import jax.numpy as jnp
from jax import lax
from jax.experimental import pallas as pl
from jax.experimental.pallas import tpu as pltpu
from jax.experimental.pallas import tpu_sc as plsc

E = 8
T = 4096
K = 2
TK = T * K
D = 1024
F = 4096
BLKLOG = 9
BLK = 1 << BLKLOG
NB = (TK + E * (BLK - 1) + BLK - 1) // BLK
P = NB * BLK
FH = F // 2
BE_PAD = 64

_info = plsc.get_sparse_core_info()
NC = _info.num_cores
NS = _info.num_subcores
L = _info.num_lanes

CHUNK = TK // NS
NV = CHUNK // L
TOKC = T // NS
NW = NC * NS
TPW = T // NW

_mesh = plsc.VectorSubcoreMesh(core_axis_name="c", subcore_axis_name="s")


@functools.partial(
    pl.kernel,
    mesh=_mesh,
    compiler_params=pltpu.CompilerParams(needs_layout_passes=False),
    out_type=[
        jax.ShapeDtypeStruct((P, D), jnp.float32),
        jax.ShapeDtypeStruct((BE_PAD,), jnp.int32),
        jax.ShapeDtypeStruct((TK,), jnp.int32),
        jax.ShapeDtypeStruct((L,), jnp.int32),
        jax.ShapeDtypeStruct((P, D), jnp.float32),
    ],
    scratch_types=[
        pltpu.VMEM((CHUNK,), jnp.int32),
        pltpu.VMEM((CHUNK,), jnp.int32),
        pltpu.VMEM((4, 64), jnp.int32),
        pltpu.VMEM((4, 64), jnp.int32),
        pltpu.VMEM((64, D), jnp.float32),
        pltpu.VMEM((L,), jnp.int32),
        pltpu.VMEM((NS, L), jnp.int32),
        pltpu.VMEM_SHARED((NS, L), jnp.int32),
        pltpu.VMEM((BE_PAD,), jnp.int32),
        pltpu.VMEM((L,), jnp.int32),
        pltpu.SemaphoreType.DMA,
    ],
)
def _route(x_hbm, fe_hbm, xg_hbm, be_hbm, pos_hbm, cnt_hbm, ybuf_hbm,
           e_chunk, posv, pe, po, xrows, hist_my, hist_all,
           hist_stage, be_v, cnt_v, sem):
    cid = lax.axis_index("c")
    sid = lax.axis_index("s")
    base = sid * CHUNK
    iota = lax.iota(jnp.int32, L)

    pltpu.sync_copy(fe_hbm.at[pl.ds(base, CHUNK)], e_chunk)

    c_acc = [jnp.int32(0) for _ in range(E)]
    for i in range(NV):
        v = e_chunk[pl.ds(i * L, L)]
        for e in range(E):
            c_acc[e] = c_acc[e] + jnp.sum((v == e).astype(jnp.int32))
    hv = jnp.zeros((L,), jnp.int32)
    for e in range(E):
        hv = jnp.where(iota == e, c_acc[e], hv)
    hist_my[...] = hv
    pltpu.sync_copy(hist_my, hist_stage.at[sid])
    plsc.subcore_barrier()
    pltpu.sync_copy(hist_stage, hist_all)

    hrow = [hist_all[s, pl.ds(0, L)] for s in range(NS)]
    counts = []
    pref = []
    for e in range(E):
        tot = hrow[0][e]
        for s in range(1, NS):
            tot = tot + hrow[s][e]
        counts.append(tot)
        pr = jnp.int32(0)
        for s in range(NS):
            pr = pr + jnp.where(jnp.int32(s) < sid, hrow[s][e], 0)
        pref.append(pr)
    padded = [((counts[e] + (BLK - 1)) >> BLKLOG) << BLKLOG for e in range(E)]
    excl = []
    cum = []
    run = jnp.int32(0)
    for e in range(E):
        excl.append(run)
        run = run + padded[e]
        cum.append(run)
    start = [excl[e] + pref[e] for e in range(E)]

    for j in range(BE_PAD // L):
        bv = (iota + j * L) * BLK
        acc = jnp.zeros((L,), jnp.int32)
        for e in range(E):
            acc = acc + (cum[e] <= bv).astype(jnp.int32)
        bev = jnp.minimum(acc, E - 1)
        if j == BE_PAD // L - 1:
            bev = jnp.where(iota == L - 1, cum[E - 1] >> BLKLOG, bev)
        be_v[pl.ds(j * L, L)] = bev
    cv = jnp.zeros((L,), jnp.int32)
    for e in range(E):
        cv = jnp.where(iota == e, counts[e], cv)
    cnt_v[...] = cv

    @pl.when(jnp.logical_and(cid == 0, sid == 0))
    def _():
        pltpu.sync_copy(be_v, be_hbm)
        pltpu.sync_copy(cnt_v, cnt_hbm)

    cur = list(start)
    for i in range(NV):
        v = e_chunk[pl.ds(i * L, L)]
        p = jnp.zeros((L,), jnp.int32)
        for e in range(E):
            m = v == e
            mi = m.astype(jnp.int32)
            cs = plsc.cumsum(mi)
            p = jnp.where(m, cur[e] + cs - 1, p)
            cur[e] = cur[e] + jnp.sum(mi)
        posv[pl.ds(i * L, L)] = p

    @pl.when(cid == 0)
    def _():
        pltpu.sync_copy(posv, pos_hbm.at[pl.ds(base, CHUNK)])

    for c4 in range(4):
        for jv in range(4):
            idx_e = c4 * 128 + 2 * (jv * L + iota)
            pe[c4, pl.ds(jv * L, L)] = plsc.load_gather(posv, [idx_e])
            po[c4, pl.ds(jv * L, L)] = plsc.load_gather(posv, [idx_e + 1])

    for c4 in range(4):
        tok0 = sid * TOKC + c4 * 64
        pltpu.sync_copy(x_hbm.at[pl.ds(tok0, 64)], xrows)
        if NC == 1:
            pltpu.async_copy(xrows, xg_hbm.at[pe.at[c4]], sem).wait()
            pltpu.async_copy(xrows, xg_hbm.at[po.at[c4]], sem).wait()
        else:
            @pl.when(cid == 0)
            def _():
                pltpu.async_copy(xrows, xg_hbm.at[pe.at[c4]], sem).wait()

            @pl.when(cid == 1)
            def _():
                pltpu.async_copy(xrows, xg_hbm.at[po.at[c4]], sem).wait()


def _mlp_body(be_ref, xg_ref, yin_ref, w1_ref, w2_ref, out_ref):
    j = pl.program_id(0)
    b = pl.program_id(1)
    used = be_ref[BE_PAD - 1]

    @pl.when(b < used)
    def _():
        xb = xg_ref[...].astype(jnp.bfloat16)
        h = jax.nn.relu(
            lax.dot_general(xb, w1_ref[0],
                            dimension_numbers=(((1,), (0,)), ((), ())),
                            preferred_element_type=jnp.float32))
        contrib = lax.dot_general(h.astype(jnp.bfloat16), w2_ref[0],
                                  dimension_numbers=(((1,), (0,)), ((), ())),
                                  preferred_element_type=jnp.float32)

        @pl.when(j == 0)
        def _():
            out_ref[...] = contrib

        @pl.when(j > 0)
        def _():
            out_ref[...] = yin_ref[...] + contrib


_mlp = pl.pallas_call(
    _mlp_body,
    grid_spec=pltpu.PrefetchScalarGridSpec(
        num_scalar_prefetch=1,
        grid=(2, NB),
        in_specs=[
            pl.BlockSpec((BLK, D), lambda j, b, be: (b, 0)),
            pl.BlockSpec((BLK, D), lambda j, b, be: (jnp.where(j > 0, b, NB - 1), 0)),
            pl.BlockSpec((1, D, FH), lambda j, b, be: (be[b], 0, j)),
            pl.BlockSpec((1, FH, D), lambda j, b, be: (be[b], j, 0)),
        ],
        out_specs=pl.BlockSpec((BLK, D), lambda j, b, be: (b, 0)),
    ),
    out_shape=jax.ShapeDtypeStruct((P, D), jnp.float32),
    input_output_aliases={2: 0},
    compiler_params=pltpu.CompilerParams(
        dimension_semantics=("arbitrary", "arbitrary")),
)


@functools.partial(
    pl.kernel,
    mesh=_mesh,
    compiler_params=pltpu.CompilerParams(needs_layout_passes=False),
    out_type=jax.ShapeDtypeStruct((T, D), jnp.float32),
    scratch_types=[
        pltpu.VMEM((2 * TPW,), jnp.int32),
        pltpu.VMEM((2 * TPW,), jnp.float32),
        pltpu.VMEM((4, 32), jnp.int32),
        pltpu.VMEM((4, 32), jnp.int32),
        pltpu.VMEM((32, D), jnp.float32),
        pltpu.VMEM((32, D), jnp.float32),
        pltpu.SemaphoreType.DMA,
    ],
)
def _combine(ew_hbm, pos_hbm, yg_hbm, out_hbm, pidx, ewv, pe, po,
             buf0, buf1, sem):
    cid = lax.axis_index("c")
    sid = lax.axis_index("s")
    wid = sid * NC + cid
    tok0 = wid * TPW
    iota = lax.iota(jnp.int32, L)

    pltpu.sync_copy(pos_hbm.at[pl.ds(2 * tok0, 2 * TPW)], pidx)
    pltpu.sync_copy(ew_hbm.at[pl.ds(2 * tok0, 2 * TPW)], ewv)

    for c in range(TPW // 32):
        for jv in range(2):
            idx_e = c * 64 + 2 * (jv * L + iota)
            pe[c, pl.ds(jv * L, L)] = plsc.load_gather(pidx, [idx_e])
            po[c, pl.ds(jv * L, L)] = plsc.load_gather(pidx, [idx_e + 1])

    for c in range(TPW // 32):
        pltpu.async_copy(yg_hbm.at[pe.at[c]], buf0, sem).wait()
        pltpu.async_copy(yg_hbm.at[po.at[c]], buf1, sem).wait()

        def body(r, _, c=c):
            g0 = jnp.full((L,), 2 * (c * 32 + r), jnp.int32)
            w0 = plsc.load_gather(ewv, [g0])
            w1 = plsc.load_gather(ewv, [g0 + 1])
            for k in range(D // L):
                a = buf0[r, pl.ds(k * L, L)]
                b = buf1[r, pl.ds(k * L, L)]
                buf0[r, pl.ds(k * L, L)] = a * w0 + b * w1
            return 0

        lax.fori_loop(0, 32, body, 0)
        pltpu.sync_copy(buf0, out_hbm.at[pl.ds(tok0 + c * 32, 32)])


def kernel(x, expert_weights, W1, W2, top_experts):
    fe = top_experts.astype(jnp.int32).reshape(TK)
    ew = expert_weights.astype(jnp.float32).reshape(TK)
    xg, be, pos, cnt, ybuf = _route(x, fe)
    yg = _mlp(be, xg, ybuf, W1, W2)
    out = _combine(ew, pos, yg)
    return (out, cnt[:E])


# speedup vs baseline: 4.2085x; 1.0104x over previous
import functools

import jax
import jax.numpy as jnp
from jax import lax
from jax.experimental import pallas as pl
from jax.experimental.pallas import tpu as pltpu
from jax.experimental.pallas import tpu_sc as plsc

E = 8
T = 4096
K = 2
TK = T * K
D = 1024
F = 4096
BLKLOG = 9
BLK = 1 << BLKLOG
NB = (TK + E * (BLK - 1) + BLK - 1) // BLK
P = NB * BLK
FH = F // 2
BE_PAD = 64

_info = plsc.get_sparse_core_info()
NC = _info.num_cores
NS = _info.num_subcores
L = _info.num_lanes

CHUNK = TK // NS
NV = CHUNK // L
TOKC = T // NS
NW = NC * NS
TPW = T // NW

_mesh = plsc.VectorSubcoreMesh(core_axis_name="c", subcore_axis_name="s")


@functools.partial(
    pl.kernel,
    mesh=_mesh,
    compiler_params=pltpu.CompilerParams(needs_layout_passes=False),
    out_type=[
        jax.ShapeDtypeStruct((P, D), jnp.float32),
        jax.ShapeDtypeStruct((BE_PAD,), jnp.int32),
        jax.ShapeDtypeStruct((TK,), jnp.int32),
        jax.ShapeDtypeStruct((L,), jnp.int32),
        jax.ShapeDtypeStruct((P, D), jnp.float32),
    ],
    scratch_types=[
        pltpu.VMEM((CHUNK,), jnp.int32),
        pltpu.VMEM((CHUNK,), jnp.int32),
        pltpu.VMEM((4, 64), jnp.int32),
        pltpu.VMEM((4, 64), jnp.int32),
        pltpu.VMEM((64, D), jnp.float32),
        pltpu.VMEM((L,), jnp.int32),
        pltpu.VMEM((NS, L), jnp.int32),
        pltpu.VMEM_SHARED((NS, L), jnp.int32),
        pltpu.VMEM((BE_PAD,), jnp.int32),
        pltpu.VMEM((L,), jnp.int32),
        pltpu.SemaphoreType.DMA,
    ],
)
def _route(x_hbm, fe_hbm, xg_hbm, be_hbm, pos_hbm, cnt_hbm, ybuf_hbm,
           e_chunk, posv, pe, po, xrows, hist_my, hist_all,
           hist_stage, be_v, cnt_v, sem):
    cid = lax.axis_index("c")
    sid = lax.axis_index("s")
    base = sid * CHUNK
    iota = lax.iota(jnp.int32, L)

    pltpu.sync_copy(fe_hbm.at[pl.ds(base, CHUNK)], e_chunk)

    c_acc = [jnp.int32(0) for _ in range(E)]
    for i in range(NV):
        v = e_chunk[pl.ds(i * L, L)]
        for e in range(E):
            c_acc[e] = c_acc[e] + jnp.sum((v == e).astype(jnp.int32))
    hv = jnp.zeros((L,), jnp.int32)
    for e in range(E):
        hv = jnp.where(iota == e, c_acc[e], hv)
    hist_my[...] = hv
    pltpu.sync_copy(hist_my, hist_stage.at[sid])
    plsc.subcore_barrier()
    pltpu.sync_copy(hist_stage, hist_all)

    hrow = [hist_all[s, pl.ds(0, L)] for s in range(NS)]
    counts = []
    pref = []
    for e in range(E):
        tot = hrow[0][e]
        for s in range(1, NS):
            tot = tot + hrow[s][e]
        counts.append(tot)
        pr = jnp.int32(0)
        for s in range(NS):
            pr = pr + jnp.where(jnp.int32(s) < sid, hrow[s][e], 0)
        pref.append(pr)
    padded = [((counts[e] + (BLK - 1)) >> BLKLOG) << BLKLOG for e in range(E)]
    excl = []
    cum = []
    run = jnp.int32(0)
    for e in range(E):
        excl.append(run)
        run = run + padded[e]
        cum.append(run)
    start = [excl[e] + pref[e] for e in range(E)]

    for j in range(BE_PAD // L):
        bv = (iota + j * L) * BLK
        acc = jnp.zeros((L,), jnp.int32)
        for e in range(E):
            acc = acc + (cum[e] <= bv).astype(jnp.int32)
        bev = jnp.minimum(acc, E - 1)
        if j == BE_PAD // L - 1:
            bev = jnp.where(iota == L - 1, cum[E - 1] >> BLKLOG, bev)
        be_v[pl.ds(j * L, L)] = bev
    cv = jnp.zeros((L,), jnp.int32)
    for e in range(E):
        cv = jnp.where(iota == e, counts[e], cv)
    cnt_v[...] = cv

    @pl.when(jnp.logical_and(cid == 0, sid == 0))
    def _():
        pltpu.sync_copy(be_v, be_hbm)
        pltpu.sync_copy(cnt_v, cnt_hbm)

    cur = list(start)
    for i in range(NV):
        v = e_chunk[pl.ds(i * L, L)]
        p = jnp.zeros((L,), jnp.int32)
        for e in range(E):
            m = v == e
            mi = m.astype(jnp.int32)
            cs = plsc.cumsum(mi)
            p = jnp.where(m, cur[e] + cs - 1, p)
            cur[e] = cur[e] + jnp.sum(mi)
        posv[pl.ds(i * L, L)] = p

    @pl.when(cid == 0)
    def _():
        pltpu.sync_copy(posv, pos_hbm.at[pl.ds(base, CHUNK)])

    for c4 in range(4):
        for jv in range(4):
            idx_e = c4 * 128 + 2 * (jv * L + iota)
            pe[c4, pl.ds(jv * L, L)] = plsc.load_gather(posv, [idx_e])
            po[c4, pl.ds(jv * L, L)] = plsc.load_gather(posv, [idx_e + 1])

    for c4 in range(4):
        tok0 = sid * TOKC + c4 * 64
        pltpu.sync_copy(x_hbm.at[pl.ds(tok0, 64)], xrows)
        if NC == 1:
            pltpu.async_copy(xrows, xg_hbm.at[pe.at[c4]], sem).wait()
            pltpu.async_copy(xrows, xg_hbm.at[po.at[c4]], sem).wait()
        else:
            @pl.when(cid == 0)
            def _():
                pltpu.async_copy(xrows, xg_hbm.at[pe.at[c4]], sem).wait()

            @pl.when(cid == 1)
            def _():
                pltpu.async_copy(xrows, xg_hbm.at[po.at[c4]], sem).wait()


def _mlp_body(be_ref, xg_ref, yin_ref, w1_ref, w2_ref, out_ref):
    j = pl.program_id(0)
    b = pl.program_id(1)
    used = be_ref[BE_PAD - 1]

    @pl.when(b < used)
    def _():
        xb = xg_ref[...].astype(jnp.bfloat16)
        h = jax.nn.relu(
            lax.dot_general(xb, w1_ref[0],
                            dimension_numbers=(((1,), (0,)), ((), ())),
                            preferred_element_type=jnp.float32))
        contrib = lax.dot_general(h.astype(jnp.bfloat16), w2_ref[0],
                                  dimension_numbers=(((1,), (0,)), ((), ())),
                                  preferred_element_type=jnp.float32)

        @pl.when(j == 0)
        def _():
            out_ref[...] = contrib

        @pl.when(j > 0)
        def _():
            out_ref[...] = yin_ref[...] + contrib


_mlp = pl.pallas_call(
    _mlp_body,
    grid_spec=pltpu.PrefetchScalarGridSpec(
        num_scalar_prefetch=1,
        grid=(2, NB),
        in_specs=[
            pl.BlockSpec((BLK, D), lambda j, b, be: (b, 0)),
            pl.BlockSpec((BLK, D), lambda j, b, be: (jnp.where(j > 0, b, NB - 1), 0)),
            pl.BlockSpec((1, D, FH), lambda j, b, be: (be[b], 0, j)),
            pl.BlockSpec((1, FH, D), lambda j, b, be: (be[b], j, 0)),
        ],
        out_specs=pl.BlockSpec((BLK, D), lambda j, b, be: (b, 0)),
    ),
    out_shape=jax.ShapeDtypeStruct((P, D), jnp.float32),
    input_output_aliases={2: 0},
    compiler_params=pltpu.CompilerParams(
        dimension_semantics=("arbitrary", "arbitrary")),
)


@functools.partial(
    pl.kernel,
    mesh=_mesh,
    compiler_params=pltpu.CompilerParams(needs_layout_passes=False),
    out_type=jax.ShapeDtypeStruct((T, D), jnp.float32),
    scratch_types=[
        pltpu.VMEM((2 * TPW,), jnp.int32),
        pltpu.VMEM((2 * TPW,), jnp.float32),
        pltpu.VMEM((4, 32), jnp.int32),
        pltpu.VMEM((4, 32), jnp.int32),
        pltpu.VMEM((32, D), jnp.float32),
        pltpu.VMEM((32, D), jnp.float32),
        pltpu.SemaphoreType.DMA,
    ],
)
def _combine(ew_hbm, pos_hbm, yg_hbm, out_hbm, pidx, ewv, pe, po,
             buf0, buf1, sem):
    cid = lax.axis_index("c")
    sid = lax.axis_index("s")
    wid = sid * NC + cid
    tok0 = wid * TPW
    iota = lax.iota(jnp.int32, L)

    pltpu.sync_copy(pos_hbm.at[pl.ds(2 * tok0, 2 * TPW)], pidx)
    pltpu.sync_copy(ew_hbm.at[pl.ds(2 * tok0, 2 * TPW)], ewv)

    for c in range(TPW // 32):
        for jv in range(2):
            idx_e = c * 64 + 2 * (jv * L + iota)
            pe[c, pl.ds(jv * L, L)] = plsc.load_gather(pidx, [idx_e])
            po[c, pl.ds(jv * L, L)] = plsc.load_gather(pidx, [idx_e + 1])

    for c in range(TPW // 32):
        d0 = pltpu.async_copy(yg_hbm.at[pe.at[c]], buf0, sem)
        d1 = pltpu.async_copy(yg_hbm.at[po.at[c]], buf1, sem)
        d0.wait()
        d1.wait()

        def body(r, _, c=c):
            g0 = jnp.full((L,), 2 * (c * 32 + r), jnp.int32)
            w0 = plsc.load_gather(ewv, [g0])
            w1 = plsc.load_gather(ewv, [g0 + 1])
            for k in range(D // L):
                a = buf0[r, pl.ds(k * L, L)]
                b = buf1[r, pl.ds(k * L, L)]
                buf0[r, pl.ds(k * L, L)] = a * w0 + b * w1
            return 0

        lax.fori_loop(0, 32, body, 0)
        pltpu.sync_copy(buf0, out_hbm.at[pl.ds(tok0 + c * 32, 32)])


def kernel(x, expert_weights, W1, W2, top_experts):
    fe = top_experts.astype(jnp.int32).reshape(TK)
    ew = expert_weights.astype(jnp.float32).reshape(TK)
    xg, be, pos, cnt, ybuf = _route(x, fe)
    yg = _mlp(be, xg, ybuf, W1, W2)
    out = _combine(ew, pos, yg)
    return (out, cnt[:E])
